# baseline (speedup 1.0000x reference)
import functools

import jax
import jax.numpy as jnp
from jax import lax
from jax.experimental import pallas as pl
from jax.experimental.pallas import tpu as pltpu
from jax.experimental.pallas import tpu_sc as plsc

NUM_NODES = 1000000
HIDDEN = 32
BATCH = 16384
LANES = 16
NW = 32
B_PER_W = BATCH // NW
CHUNK = 128
NCHUNK = B_PER_W // CHUNK
GROUPS = B_PER_W // LANES
EPS2 = 1e-24


def _rsqrt(x):
    i = plsc.bitcast(x, jnp.int32)
    i = jnp.int32(0x5F3759DF) - lax.shift_right_arithmetic(i, 1)
    y = plsc.bitcast(i, jnp.float32)
    for _ in range(3):
        y = y * (1.5 - 0.5 * x * y * y)
    return y


def _make_sc_kernel():
    mesh = plsc.VectorSubcoreMesh(core_axis_name="c", subcore_axis_name="s")

    @functools.partial(
        pl.kernel,
        mesh=mesh,
        compiler_params=pltpu.CompilerParams(
            needs_layout_passes=False, use_tc_tiling_on_sc=False),
        out_type=jax.ShapeDtypeStruct((BATCH,), jnp.float32),
        scratch_types=[
            pltpu.VMEM((NCHUNK, CHUNK), jnp.int32),
            pltpu.VMEM((NCHUNK, CHUNK), jnp.int32),
            pltpu.VMEM((NCHUNK, CHUNK), jnp.int32),
            pltpu.VMEM((B_PER_W, HIDDEN), jnp.float32),
            pltpu.VMEM((B_PER_W, HIDDEN), jnp.float32),
            pltpu.VMEM((B_PER_W, HIDDEN), jnp.float32),
            pltpu.VMEM((B_PER_W,), jnp.float32),
            pltpu.SemaphoreType.DMA,
            pltpu.SemaphoreType.DMA((NCHUNK,)),
        ],
    )
    def scored(heads_hbm, rels_hbm, tails_hbm, node_hbm, rel_hbm, out_hbm,
               hidx, ridx, tidx, hrows, rrows, trows, score_v, sem, sems):
        wid = lax.axis_index("s") * 2 + lax.axis_index("c")
        base = wid * B_PER_W

        idx_copies = [
            pltpu.async_copy(heads_hbm.at[pl.ds(wid * NCHUNK, NCHUNK)], hidx, sem),
            pltpu.async_copy(rels_hbm.at[pl.ds(wid * NCHUNK, NCHUNK)], ridx, sem),
            pltpu.async_copy(tails_hbm.at[pl.ds(wid * NCHUNK, NCHUNK)], tidx, sem),
        ]
        for c in idx_copies:
            c.wait()

        copies = []
        for j in range(NCHUNK):
            dst = pl.ds(j * CHUNK, CHUNK)
            copies.append((
                pltpu.async_copy(node_hbm.at[hidx.at[j]], hrows.at[dst], sems.at[j]),
                pltpu.async_copy(rel_hbm.at[ridx.at[j]], rrows.at[dst], sems.at[j]),
                pltpu.async_copy(node_hbm.at[tidx.at[j]], trows.at[dst], sems.at[j]),
            ))

        lane = lax.iota(jnp.int32, LANES)

        def group_body(g, carry):
            row = g * LANES + lane
            zero = jnp.zeros((LANES,), jnp.float32)
            hh = zero
            tt = zero
            rr = zero
            hr = zero
            ht = zero
            rt = zero
            for d in range(HIDDEN):
                col = jnp.full((LANES,), d, jnp.int32)
                hv = plsc.load_gather(hrows, [row, col])
                rv = plsc.load_gather(rrows, [row, col])
                tv = plsc.load_gather(trows, [row, col])
                hh = hh + hv * hv
                tt = tt + tv * tv
                rr = rr + rv * rv
                hr = hr + hv * rv
                ht = ht + hv * tv
                rt = rt + rv * tv
            inh = _rsqrt(jnp.maximum(hh, EPS2))
            int_ = _rsqrt(jnp.maximum(tt, EPS2))
            d2 = (hh * inh * inh + tt * int_ * int_ + rr
                  + 2.0 * hr * inh - 2.0 * ht * (inh * int_) - 2.0 * rt * int_)
            d2 = jnp.maximum(d2, 0.0)
            score = -(d2 * _rsqrt(jnp.maximum(d2, 1e-30)))
            score_v[pl.ds(g * LANES, LANES)] = score
            return carry

        gpc = GROUPS // NCHUNK
        for j in range(NCHUNK):
            for c in copies[j]:
                c.wait()
            lax.fori_loop(j * gpc, (j + 1) * gpc, group_body, 0)

        pltpu.sync_copy(score_v, out_hbm.at[pl.ds(base, B_PER_W)])

    return scored


_sc_score = _make_sc_kernel()


def kernel(batched_paths, node_emb, rel_emb):
    heads = batched_paths[:, 2].reshape(NW * NCHUNK, CHUNK)
    rels = batched_paths[:, 1].reshape(NW * NCHUNK, CHUNK)
    tails = batched_paths[:, 0].reshape(NW * NCHUNK, CHUNK)
    return _sc_score(heads, rels, tails, node_emb, rel_emb)


# speedup vs baseline: 1.4718x; 1.4718x over previous
import functools

import jax
import jax.numpy as jnp
from jax import lax
from jax.experimental import pallas as pl
from jax.experimental.pallas import tpu as pltpu
from jax.experimental.pallas import tpu_sc as plsc

HIDDEN = 32
BATCH = 16384
LANES = 16
NW = 32
B_PER_W = BATCH // NW
CHUNK = 128
NCHUNK = B_PER_W // CHUNK
GPC = CHUNK // LANES
EPS2 = 1e-24


def _rsqrt(x):
    i = plsc.bitcast(x, jnp.int32)
    i = jnp.int32(0x5F3759DF) - lax.shift_right_arithmetic(i, 1)
    y = plsc.bitcast(i, jnp.float32)
    for _ in range(3):
        y = y * (1.5 - 0.5 * x * y * y)
    return y


def _make_sc_kernel():
    mesh = plsc.VectorSubcoreMesh(core_axis_name="c", subcore_axis_name="s")
    rowbuf = pltpu.VMEM((CHUNK, HIDDEN), jnp.float32)

    @functools.partial(
        pl.kernel,
        mesh=mesh,
        compiler_params=pltpu.CompilerParams(needs_layout_passes=False),
        out_type=jax.ShapeDtypeStruct((BATCH,), jnp.float32),
        scratch_types=[
            pltpu.VMEM((B_PER_W,), jnp.int32),
            pltpu.VMEM((B_PER_W,), jnp.int32),
            pltpu.VMEM((B_PER_W,), jnp.int32),
            rowbuf, rowbuf,
            rowbuf, rowbuf,
            rowbuf, rowbuf,
            pltpu.VMEM((B_PER_W,), jnp.float32),
            pltpu.SemaphoreType.DMA,
            pltpu.SemaphoreType.DMA((2,)),
        ],
    )
    def scored(heads_hbm, rels_hbm, tails_hbm, node_hbm, rel_hbm, out_hbm,
               hidx_v, ridx_v, tidx_v,
               h0, h1, r0, r1, t0, t1, score_v, isem, sems):
        wid = lax.axis_index("s") * 2 + lax.axis_index("c")
        base = wid * B_PER_W
        sl = pl.ds(base, B_PER_W)
        hbuf = (h0, h1)
        rbuf = (r0, r1)
        tbuf = (t0, t1)

        v_copies = [
            pltpu.async_copy(heads_hbm.at[sl], hidx_v, isem),
            pltpu.async_copy(rels_hbm.at[sl], ridx_v, isem),
            pltpu.async_copy(tails_hbm.at[sl], tidx_v, isem),
        ]
        for c in v_copies:
            c.wait()

        def fire(j):
            p = j % 2
            hb, rb, tb = hbuf[p], rbuf[p], tbuf[p]

            def body(i, carry):
                src = pl.ds(j * CHUNK + i * LANES, LANES)
                hvec = hidx_v[src]
                rvec = ridx_v[src]
                tvec = tidx_v[src]
                for k in range(LANES):
                    dst = pl.ds(i * LANES + k, 1)
                    pltpu.async_copy(
                        node_hbm.at[pl.ds(hvec[k], 1), :],
                        hb.at[dst, :], sems.at[p])
                    pltpu.async_copy(
                        rel_hbm.at[pl.ds(rvec[k], 1), :],
                        rb.at[dst, :], sems.at[p])
                    pltpu.async_copy(
                        node_hbm.at[pl.ds(tvec[k], 1), :],
                        tb.at[dst, :], sems.at[p])
                return carry

            lax.fori_loop(0, CHUNK // LANES, body, 0)

        def drain(j):
            p = j % 2
            src = node_hbm.at[pl.ds(0, CHUNK), :]
            pltpu.make_async_copy(src, hbuf[p], sems.at[p]).wait()
            pltpu.make_async_copy(src, rbuf[p], sems.at[p]).wait()
            pltpu.make_async_copy(src, tbuf[p], sems.at[p]).wait()

        lane = lax.iota(jnp.int32, LANES)

        def make_group_body(hb, rb, tb, out_base):
            def group_body(g, carry):
                row = g * LANES + lane
                zero = jnp.zeros((LANES,), jnp.float32)
                hh = zero
                tt = zero
                rr = zero
                hr = zero
                ht = zero
                rt = zero
                for d in range(HIDDEN):
                    col = jnp.full((LANES,), d, jnp.int32)
                    hv = plsc.load_gather(hb, [row, col])
                    rv = plsc.load_gather(rb, [row, col])
                    tv = plsc.load_gather(tb, [row, col])
                    hh = hh + hv * hv
                    tt = tt + tv * tv
                    rr = rr + rv * rv
                    hr = hr + hv * rv
                    ht = ht + hv * tv
                    rt = rt + rv * tv
                inh = _rsqrt(jnp.maximum(hh, EPS2))
                int_ = _rsqrt(jnp.maximum(tt, EPS2))
                d2 = (hh * inh * inh + tt * int_ * int_ + rr
                      + 2.0 * hr * inh - 2.0 * ht * (inh * int_)
                      - 2.0 * rt * int_)
                d2 = jnp.maximum(d2, 0.0)
                score = -(d2 * _rsqrt(jnp.maximum(d2, 1e-30)))
                score_v[pl.ds(out_base + g * LANES, LANES)] = score
                return carry

            return group_body

        fire(0)
        for j in range(NCHUNK):
            drain(j)
            if j + 1 < NCHUNK:
                fire(j + 1)
            p = j % 2
            body = make_group_body(hbuf[p], rbuf[p], tbuf[p], j * CHUNK)
            lax.fori_loop(0, GPC, body, 0)

        pltpu.sync_copy(score_v, out_hbm.at[sl])

    return scored


_sc_score = _make_sc_kernel()


def kernel(batched_paths, node_emb, rel_emb):
    heads = batched_paths[:, 2]
    rels = batched_paths[:, 1]
    tails = batched_paths[:, 0]
    return _sc_score(heads, rels, tails, node_emb, rel_emb)


# speedup vs baseline: 1.4739x; 1.0014x over previous
import functools

import jax
import jax.numpy as jnp
from jax import lax
from jax.experimental import pallas as pl
from jax.experimental.pallas import tpu as pltpu
from jax.experimental.pallas import tpu_sc as plsc

HIDDEN = 32
BATCH = 16384
LANES = 16
NW = 32
B_PER_W = BATCH // NW
CHUNK = 128
NCHUNK = B_PER_W // CHUNK
GPC = CHUNK // LANES
EPS2 = 1e-24


def _rsqrt(x):
    i = plsc.bitcast(x, jnp.int32)
    i = jnp.int32(0x5F3759DF) - lax.shift_right_arithmetic(i, 1)
    y = plsc.bitcast(i, jnp.float32)
    for _ in range(3):
        y = y * (1.5 - 0.5 * x * y * y)
    return y


def _make_sc_kernel():
    mesh = plsc.VectorSubcoreMesh(core_axis_name="c", subcore_axis_name="s")
    rowbuf = pltpu.VMEM((CHUNK, HIDDEN), jnp.float32)

    @functools.partial(
        pl.kernel,
        mesh=mesh,
        compiler_params=pltpu.CompilerParams(needs_layout_passes=False),
        out_type=jax.ShapeDtypeStruct((BATCH,), jnp.float32),
        scratch_types=[
            pltpu.VMEM((B_PER_W,), jnp.int32),
            pltpu.VMEM((B_PER_W,), jnp.int32),
            pltpu.VMEM((B_PER_W,), jnp.int32),
            rowbuf, rowbuf,
            rowbuf, rowbuf,
            rowbuf, rowbuf,
            pltpu.VMEM((B_PER_W,), jnp.float32),
            pltpu.SemaphoreType.DMA,
            pltpu.SemaphoreType.DMA((2, 8)),
        ],
    )
    def scored(heads_hbm, rels_hbm, tails_hbm, node_hbm, rel_hbm, out_hbm,
               hidx_v, ridx_v, tidx_v,
               h0, h1, r0, r1, t0, t1, score_v, isem, sems):
        wid = lax.axis_index("s") * 2 + lax.axis_index("c")
        base = wid * B_PER_W
        sl = pl.ds(base, B_PER_W)
        hbuf = (h0, h1)
        rbuf = (r0, r1)
        tbuf = (t0, t1)

        v_copies = [
            pltpu.async_copy(heads_hbm.at[sl], hidx_v, isem),
            pltpu.async_copy(rels_hbm.at[sl], ridx_v, isem),
            pltpu.async_copy(tails_hbm.at[sl], tidx_v, isem),
        ]
        for c in v_copies:
            c.wait()

        def fire(j):
            p = j % 2
            hb, rb, tb = hbuf[p], rbuf[p], tbuf[p]

            def body(i, carry):
                src = pl.ds(j * CHUNK + i * LANES, LANES)
                hvec = hidx_v[src]
                rvec = ridx_v[src]
                tvec = tidx_v[src]
                for k in range(LANES):
                    dst = pl.ds(i * LANES + k, 1)
                    q = k % 8
                    pltpu.async_copy(
                        node_hbm.at[pl.ds(hvec[k], 1), :],
                        hb.at[dst, :], sems.at[p, q])
                    pltpu.async_copy(
                        rel_hbm.at[pl.ds(rvec[k], 1), :],
                        rb.at[dst, :], sems.at[p, q])
                    pltpu.async_copy(
                        node_hbm.at[pl.ds(tvec[k], 1), :],
                        tb.at[dst, :], sems.at[p, q])
                return carry

            lax.fori_loop(0, CHUNK // LANES, body, 0)

        def drain(j):
            p = j % 2
            src = node_hbm.at[pl.ds(0, CHUNK // 8), :]
            for q in range(8):
                for buf in (hbuf[p], rbuf[p], tbuf[p]):
                    pltpu.make_async_copy(
                        src, buf.at[pl.ds(0, CHUNK // 8), :],
                        sems.at[p, q]).wait()

        lane = lax.iota(jnp.int32, LANES)

        def make_group_body(hb, rb, tb, out_base):
            def group_body(g, carry):
                row = g * LANES + lane
                zero = jnp.zeros((LANES,), jnp.float32)
                hh = zero
                tt = zero
                rr = zero
                hr = zero
                ht = zero
                rt = zero
                for d in range(HIDDEN):
                    col = jnp.full((LANES,), d, jnp.int32)
                    hv = plsc.load_gather(hb, [row, col])
                    rv = plsc.load_gather(rb, [row, col])
                    tv = plsc.load_gather(tb, [row, col])
                    hh = hh + hv * hv
                    tt = tt + tv * tv
                    rr = rr + rv * rv
                    hr = hr + hv * rv
                    ht = ht + hv * tv
                    rt = rt + rv * tv
                inh = _rsqrt(jnp.maximum(hh, EPS2))
                int_ = _rsqrt(jnp.maximum(tt, EPS2))
                d2 = (hh * inh * inh + tt * int_ * int_ + rr
                      + 2.0 * hr * inh - 2.0 * ht * (inh * int_)
                      - 2.0 * rt * int_)
                d2 = jnp.maximum(d2, 0.0)
                score = -(d2 * _rsqrt(jnp.maximum(d2, 1e-30)))
                score_v[pl.ds(out_base + g * LANES, LANES)] = score
                return carry

            return group_body

        fire(0)
        for j in range(NCHUNK):
            drain(j)
            if j + 1 < NCHUNK:
                fire(j + 1)
            p = j % 2
            body = make_group_body(hbuf[p], rbuf[p], tbuf[p], j * CHUNK)
            lax.fori_loop(0, GPC, body, 0)

        pltpu.sync_copy(score_v, out_hbm.at[sl])

    return scored


_sc_score = _make_sc_kernel()


def kernel(batched_paths, node_emb, rel_emb):
    heads = batched_paths[:, 2]
    rels = batched_paths[:, 1]
    tails = batched_paths[:, 0]
    return _sc_score(heads, rels, tails, node_emb, rel_emb)
